# baseline (speedup 1.0000x reference)
import functools

import jax
import jax.numpy as jnp
from jax import lax
from jax.experimental import pallas as pl
from jax.experimental.pallas import tpu as pltpu
from jax.experimental.pallas import tpu_sc as plsc

EMB_SIZE = 1000000
EMB_UNITS = 16
OUT_UNITS = 128
B = 16384
F = 26

NC, NS = 2, 16
NW = NC * NS
N_IDX = B * F
PER_W = N_IDX // NW
CHUNK = 1024
NCHUNK = PER_W // CHUNK

_mesh = plsc.VectorSubcoreMesh(
    core_axis_name="c", subcore_axis_name="s", num_cores=NC, num_subcores=NS
)
_sc_params = pltpu.CompilerParams(use_tc_tiling_on_sc=False)

DCB = 65536


def _detile_body(x_ref, *o_refs):
    for u in range(EMB_UNITS):
        o_refs[u][...] = x_ref[u, :]


def _detile(embT):
    grid = (EMB_SIZE + DCB - 1) // DCB
    return pl.pallas_call(
        _detile_body,
        grid=(grid,),
        in_specs=[pl.BlockSpec((EMB_UNITS, DCB), lambda c: (0, c))],
        out_specs=[pl.BlockSpec((DCB,), lambda c: (c,))] * EMB_UNITS,
        out_shape=[jax.ShapeDtypeStruct((EMB_SIZE,), jnp.float32)] * EMB_UNITS,
    )(embT)


RC = 2000
NCH = EMB_SIZE // RC
KFULL = NCH // NW
KTAIL = NCH - KFULL * NW


@functools.partial(
    pl.kernel,
    out_type=jax.ShapeDtypeStruct((EMB_SIZE * EMB_UNITS,), jnp.float32),
    mesh=_mesh,
    scratch_types=[
        pltpu.VMEM((EMB_UNITS * RC,), jnp.float32),
        pltpu.VMEM((EMB_UNITS * RC,), jnp.float32),
        pltpu.VMEM((RC * EMB_UNITS,), jnp.float32),
        pltpu.VMEM((RC * EMB_UNITS,), jnp.float32),
        pltpu.SemaphoreType.DMA,
        pltpu.SemaphoreType.DMA,
        pltpu.SemaphoreType.DMA,
        pltpu.SemaphoreType.DMA,
    ],
    compiler_params=pltpu.CompilerParams(
        use_tc_tiling_on_sc=False, needs_layout_passes=False
    ),
)
def _sc_transpose(*args):
    planes = args[:EMB_UNITS]
    out_hbm = args[EMB_UNITS]
    ibufs = args[EMB_UNITS + 1:EMB_UNITS + 3]
    obufs = args[EMB_UNITS + 3:EMB_UNITS + 5]
    isems = args[EMB_UNITS + 5:EMB_UNITS + 7]
    osems = args[EMB_UNITS + 7:EMB_UNITS + 9]
    wid = lax.axis_index("s") * NC + lax.axis_index("c")

    def fire_in(k, slot):
        row0 = (wid + NW * k) * RC
        return [
            pltpu.async_copy(
                planes[u].at[pl.ds(row0, RC)],
                ibufs[slot].at[pl.ds(u * RC, RC)],
                isems[slot],
            )
            for u in range(EMB_UNITS)
        ]

    def compute(slot):
        flat16 = lax.iota(jnp.int32, 16) * EMB_UNITS
        idx_u = [flat16 + u for u in range(EMB_UNITS)]
        grp = 16 * EMB_UNITS

        def g_body(g, carry):
            r0 = g * 16
            window = obufs[slot].at[pl.ds(r0 * EMB_UNITS, grp)]
            for u in range(EMB_UNITS):
                v = ibufs[slot][pl.ds(u * RC + r0, 16)]
                plsc.store_scatter(window, [idx_u[u]], v)
            return carry

        lax.fori_loop(0, RC // 16, g_body, 0, unroll=2)

    def fire_out(k, slot):
        row0 = (wid + NW * k) * RC
        return pltpu.async_copy(
            obufs[slot],
            out_hbm.at[pl.ds(row0 * EMB_UNITS, RC * EMB_UNITS)],
            osems[slot],
        )

    in_d = [None, None]
    out_d = [None, None]
    in_d[0] = fire_in(0, 0)
    for k in range(KFULL):
        slot = k % 2
        if k + 1 < KFULL:
            in_d[1 - slot] = fire_in(k + 1, 1 - slot)
        for d in in_d[slot]:
            d.wait()
        if k >= 2:
            out_d[slot].wait()
        compute(slot)
        out_d[slot] = fire_out(k, slot)
    out_d[0].wait()
    out_d[1].wait()

    @pl.when(wid < KTAIL)
    def _():
        for d in fire_in(KFULL, 0):
            d.wait()
        compute(0)
        fire_out(KFULL, 0).wait()


@functools.partial(
    pl.kernel,
    out_type=jax.ShapeDtypeStruct((N_IDX, EMB_UNITS), jnp.float32),
    mesh=_mesh,
    scratch_types=[
        pltpu.VMEM((PER_W,), jnp.int32),
        pltpu.VMEM((CHUNK, EMB_UNITS), jnp.float32),
        pltpu.VMEM((CHUNK, EMB_UNITS), jnp.float32),
        pltpu.SemaphoreType.DMA,
        pltpu.SemaphoreType.DMA,
    ],
    compiler_params=_sc_params,
)
def _sc_gather(idx_hbm, emb_hbm, out_hbm, idx_v, buf0, buf1, sem0, sem1):
    wid = lax.axis_index("s") * NC + lax.axis_index("c")
    base = wid * PER_W
    pltpu.sync_copy(idx_hbm.at[pl.ds(base, PER_W)], idx_v)
    bufs = (buf0, buf1)
    sems = (sem0, sem1)
    cps = [None, None]
    for k in range(NCHUNK):
        j = k % 2
        cps[j] = pltpu.async_copy(
            emb_hbm.at[idx_v.at[pl.ds(k * CHUNK, CHUNK)]], bufs[j], sems[j]
        )
        if k > 0:
            cps[1 - j].wait()
            pltpu.sync_copy(
                bufs[1 - j], out_hbm.at[pl.ds(base + (k - 1) * CHUNK, CHUNK)]
            )
    j = (NCHUNK - 1) % 2
    cps[j].wait()
    pltpu.sync_copy(bufs[j], out_hbm.at[pl.ds(base + (NCHUNK - 1) * CHUNK, CHUNK)])


BLK = 2048


def _tc_body(e_ref, w_ref, p_ref, o_ref):
    h = jnp.dot(e_ref[...], w_ref[...], preferred_element_type=jnp.float32)
    h = h + p_ref[0, :]
    mu = jnp.mean(h, axis=-1, keepdims=True)
    var = jnp.mean((h - mu) * (h - mu), axis=-1, keepdims=True)
    hn = (h - mu) * lax.rsqrt(var + 1e-5)
    y = hn * p_ref[1, :] + p_ref[2, :]
    o_ref[...] = y * jax.nn.sigmoid(y)


def kernel(x, emb, W, b, gamma, beta):
    idx = x.astype(jnp.int32).reshape(-1)
    planes = _detile(emb.T)
    emb_rm = _sc_transpose(*planes).reshape(EMB_SIZE, EMB_UNITS)
    e2 = _sc_gather(idx, emb_rm)
    e = e2.reshape(B, F * EMB_UNITS)
    params = jnp.stack([b, gamma, beta])
    y = pl.pallas_call(
        _tc_body,
        grid=(B // BLK,),
        in_specs=[
            pl.BlockSpec((BLK, F * EMB_UNITS), lambda i: (i, 0)),
            pl.BlockSpec((F * EMB_UNITS, OUT_UNITS), lambda i: (0, 0)),
            pl.BlockSpec((3, OUT_UNITS), lambda i: (0, 0)),
        ],
        out_specs=pl.BlockSpec((BLK, OUT_UNITS), lambda i: (i, 0)),
        out_shape=jax.ShapeDtypeStruct((B, OUT_UNITS), jnp.float32),
    )(e, W, params)
    return y


# speedup vs baseline: 1.0051x; 1.0051x over previous
import functools

import jax
import jax.numpy as jnp
from jax import lax
from jax.experimental import pallas as pl
from jax.experimental.pallas import tpu as pltpu
from jax.experimental.pallas import tpu_sc as plsc

EMB_SIZE = 1000000
EMB_UNITS = 16
OUT_UNITS = 128
B = 16384
F = 26

NC, NS = 2, 16
NW = NC * NS
N_IDX = B * F
PER_W = N_IDX // NW
CHUNK = 1024
NCHUNK = PER_W // CHUNK

_mesh = plsc.VectorSubcoreMesh(
    core_axis_name="c", subcore_axis_name="s", num_cores=NC, num_subcores=NS
)
_sc_params = pltpu.CompilerParams(use_tc_tiling_on_sc=False)

DCB = 65536


def _detile_body(x_ref, *o_refs):
    for u in range(EMB_UNITS):
        o_refs[u][...] = x_ref[u, :]


def _detile(embT):
    grid = (EMB_SIZE + DCB - 1) // DCB
    return pl.pallas_call(
        _detile_body,
        grid=(grid,),
        in_specs=[pl.BlockSpec((EMB_UNITS, DCB), lambda c: (0, c))],
        out_specs=[pl.BlockSpec((DCB,), lambda c: (c,))] * EMB_UNITS,
        out_shape=[jax.ShapeDtypeStruct((EMB_SIZE,), jnp.float32)] * EMB_UNITS,
    )(embT)


RC = 1600
NCH = EMB_SIZE // RC
KFULL = NCH // NW
KTAIL = NCH - KFULL * NW


@functools.partial(
    pl.kernel,
    out_type=jax.ShapeDtypeStruct((EMB_SIZE * EMB_UNITS,), jnp.float32),
    mesh=_mesh,
    scratch_types=[
        pltpu.VMEM((EMB_UNITS * RC,), jnp.float32),
        pltpu.VMEM((EMB_UNITS * RC,), jnp.float32),
        pltpu.VMEM((RC * EMB_UNITS,), jnp.float32),
        pltpu.VMEM((RC * EMB_UNITS,), jnp.float32),
        pltpu.SemaphoreType.DMA,
        pltpu.SemaphoreType.DMA,
        pltpu.SemaphoreType.DMA,
        pltpu.SemaphoreType.DMA,
    ],
    compiler_params=pltpu.CompilerParams(
        use_tc_tiling_on_sc=False, needs_layout_passes=False
    ),
)
def _sc_transpose(*args):
    planes = args[:EMB_UNITS]
    out_hbm = args[EMB_UNITS]
    ibufs = args[EMB_UNITS + 1:EMB_UNITS + 3]
    obufs = args[EMB_UNITS + 3:EMB_UNITS + 5]
    isems = args[EMB_UNITS + 5:EMB_UNITS + 7]
    osems = args[EMB_UNITS + 7:EMB_UNITS + 9]
    wid = lax.axis_index("s") * NC + lax.axis_index("c")

    def fire_in(k, slot):
        row0 = (wid + NW * k) * RC
        return [
            pltpu.async_copy(
                planes[u].at[pl.ds(row0, RC)],
                ibufs[slot].at[pl.ds(u * RC, RC)],
                isems[slot],
            )
            for u in range(EMB_UNITS)
        ]

    def compute(slot):
        flat16 = lax.iota(jnp.int32, 16) * EMB_UNITS
        idx_u = [flat16 + u for u in range(EMB_UNITS)]
        grp = 16 * EMB_UNITS

        def g_body(g, carry):
            r0 = g * 16
            window = obufs[slot].at[pl.ds(r0 * EMB_UNITS, grp)]
            for u in range(EMB_UNITS):
                v = ibufs[slot][pl.ds(u * RC + r0, 16)]
                plsc.store_scatter(window, [idx_u[u]], v)
            return carry

        lax.fori_loop(0, RC // 16, g_body, 0, unroll=2)

    def fire_out(k, slot):
        row0 = (wid + NW * k) * RC
        return pltpu.async_copy(
            obufs[slot],
            out_hbm.at[pl.ds(row0 * EMB_UNITS, RC * EMB_UNITS)],
            osems[slot],
        )

    in_d = [None, None]
    out_d = [None, None]
    in_d[0] = fire_in(0, 0)
    for k in range(KFULL):
        slot = k % 2
        if k + 1 < KFULL:
            in_d[1 - slot] = fire_in(k + 1, 1 - slot)
        for d in in_d[slot]:
            d.wait()
        if k >= 2:
            out_d[slot].wait()
        compute(slot)
        out_d[slot] = fire_out(k, slot)
    out_d[0].wait()
    out_d[1].wait()

    @pl.when(wid < KTAIL)
    def _():
        for d in fire_in(KFULL, 0):
            d.wait()
        compute(0)
        fire_out(KFULL, 0).wait()


@functools.partial(
    pl.kernel,
    out_type=jax.ShapeDtypeStruct((N_IDX, EMB_UNITS), jnp.float32),
    mesh=_mesh,
    scratch_types=[
        pltpu.VMEM((PER_W,), jnp.int32),
        pltpu.VMEM((CHUNK, EMB_UNITS), jnp.float32),
        pltpu.VMEM((CHUNK, EMB_UNITS), jnp.float32),
        pltpu.SemaphoreType.DMA,
        pltpu.SemaphoreType.DMA,
    ],
    compiler_params=_sc_params,
)
def _sc_gather(idx_hbm, emb_hbm, out_hbm, idx_v, buf0, buf1, sem0, sem1):
    wid = lax.axis_index("s") * NC + lax.axis_index("c")
    base = wid * PER_W
    pltpu.sync_copy(idx_hbm.at[pl.ds(base, PER_W)], idx_v)
    bufs = (buf0, buf1)
    sems = (sem0, sem1)
    cps = [None, None]
    for k in range(NCHUNK):
        j = k % 2
        cps[j] = pltpu.async_copy(
            emb_hbm.at[idx_v.at[pl.ds(k * CHUNK, CHUNK)]], bufs[j], sems[j]
        )
        if k > 0:
            cps[1 - j].wait()
            pltpu.sync_copy(
                bufs[1 - j], out_hbm.at[pl.ds(base + (k - 1) * CHUNK, CHUNK)]
            )
    j = (NCHUNK - 1) % 2
    cps[j].wait()
    pltpu.sync_copy(bufs[j], out_hbm.at[pl.ds(base + (NCHUNK - 1) * CHUNK, CHUNK)])


BLK = 2048


def _tc_body(e_ref, w_ref, p_ref, o_ref):
    h = jnp.dot(e_ref[...], w_ref[...], preferred_element_type=jnp.float32)
    h = h + p_ref[0, :]
    mu = jnp.mean(h, axis=-1, keepdims=True)
    var = jnp.mean((h - mu) * (h - mu), axis=-1, keepdims=True)
    hn = (h - mu) * lax.rsqrt(var + 1e-5)
    y = hn * p_ref[1, :] + p_ref[2, :]
    o_ref[...] = y * jax.nn.sigmoid(y)


def kernel(x, emb, W, b, gamma, beta):
    idx = x.astype(jnp.int32).reshape(-1)
    planes = _detile(emb.T)
    emb_rm = _sc_transpose(*planes).reshape(EMB_SIZE, EMB_UNITS)
    e2 = _sc_gather(idx, emb_rm)
    e = e2.reshape(B, F * EMB_UNITS)
    params = jnp.stack([b, gamma, beta])
    y = pl.pallas_call(
        _tc_body,
        grid=(B // BLK,),
        in_specs=[
            pl.BlockSpec((BLK, F * EMB_UNITS), lambda i: (i, 0)),
            pl.BlockSpec((F * EMB_UNITS, OUT_UNITS), lambda i: (0, 0)),
            pl.BlockSpec((3, OUT_UNITS), lambda i: (0, 0)),
        ],
        out_specs=pl.BlockSpec((BLK, OUT_UNITS), lambda i: (i, 0)),
        out_shape=jax.ShapeDtypeStruct((B, OUT_UNITS), jnp.float32),
    )(e, W, params)
    return y
